# baseline (speedup 1.0000x reference)
import functools

import jax
import jax.numpy as jnp
from jax import lax
from jax.experimental import pallas as pl
from jax.experimental.pallas import tpu as pltpu
from jax.experimental.pallas import tpu_sc as plsc

REL_LIST = ["pp", "fp", "ep", "pf", "ef", "ff", "pe", "fe"]
REL_SRC_DST = {
    "pp": ("p", "p"), "fp": ("f", "p"), "ep": ("e", "p"),
    "pf": ("p", "f"), "ef": ("e", "f"), "ff": ("f", "f"),
    "pe": ("p", "e"), "fe": ("f", "e"),
}

HID = 64
BLK = 128
G = 4
GB = G * BLK
OUT_ROWS = 25088
TRASH = 128
ACC_ROWS = OUT_ROWS + TRASH
PER_TILE_ACC = ACC_ROWS // 16
PER_TILE_OUT = OUT_ROWS // 16
PAD_DST = 1 << 29

_MESH = dict(core_axis_name="c", subcore_axis_name="s", num_cores=2,
             num_subcores=16)
_SC_PARAMS = pltpu.CompilerParams(use_tc_tiling_on_sc=False)


GRP = 4
ITER_BLKS = 2 * GRP


def _make_agg(e_pad, split_edges):
    n_tiles = 32 if split_edges else 16
    chunk = e_pad // n_tiles
    n_blk = chunk // BLK
    assert n_blk % ITER_BLKS == 0
    n_iter = n_blk // ITER_BLKS
    mesh = plsc.VectorSubcoreMesh(**_MESH)

    @functools.partial(
        pl.kernel,
        out_type=(jax.ShapeDtypeStruct((2, OUT_ROWS, HID), jnp.float32),
                  jax.ShapeDtypeStruct((2 * OUT_ROWS,), jnp.float32)),
        mesh=mesh,
        compiler_params=_SC_PARAMS,
        scratch_types=[
            pltpu.VMEM((2, GRP, 2 * BLK), jnp.int32),
            pltpu.VMEM((2, BLK), jnp.int32),
            pltpu.VMEM((2, BLK, HID), jnp.float32),
            pltpu.VMEM((BLK,), jnp.float32),
            pltpu.VMEM((PER_TILE_ACC,), jnp.float32),
            pltpu.VMEM_SHARED((ACC_ROWS, HID), jnp.float32),
            pltpu.VMEM_SHARED((ACC_ROWS,), jnp.float32),
            pltpu.SemaphoreType.DMA,
            pltpu.SemaphoreType.DMA,
        ],
    )
    def agg(eib_hbm, t_hbm, out_hbm, cnt_hbm, ebuf, idxw, rows, ones_v,
            zflat, acc_sh, cnt_sh, sem0, sem1):
        cid = lax.axis_index("c")
        sid = lax.axis_index("s")
        sems = (sem0, sem1)

        def zrow(r, _):
            for k in range(HID // 16):
                rows[0, r, pl.ds(k * 16, 16)] = jnp.zeros((16,), jnp.float32)
            return 0
        lax.fori_loop(0, BLK, zrow, 0)
        for j in range(BLK // 16):
            ones_v[pl.ds(j * 16, 16)] = jnp.ones((16,), jnp.float32)
        def zf(i, _):
            zflat[pl.ds(i * 16, 16)] = jnp.zeros((16,), jnp.float32)
            return 0
        lax.fori_loop(0, PER_TILE_ACC // 16, zf, 0)
        if PER_TILE_ACC % 16:
            zflat[pl.ds(PER_TILE_ACC - 16, 16)] = jnp.zeros((16,),
                                                            jnp.float32)
        r0 = sid * PER_TILE_ACC
        def zacc(k, _):
            pltpu.sync_copy(rows.at[0],
                            acc_sh.at[pl.ds(r0 + k * BLK, BLK), :])
            return 0
        lax.fori_loop(0, PER_TILE_ACC // BLK, zacc, 0)
        rem = PER_TILE_ACC % BLK
        if rem:
            pltpu.sync_copy(
                rows.at[0, pl.ds(0, rem), :],
                acc_sh.at[pl.ds(r0 + PER_TILE_ACC - rem, rem), :])
        pltpu.sync_copy(zflat, cnt_sh.at[pl.ds(r0, PER_TILE_ACC)])
        plsc.subcore_barrier()

        if split_edges:
            base = jnp.int32(0)
            tid = cid * 16 + sid
        else:
            base = cid * OUT_ROWS
            tid = sid
        line0 = tid * n_blk
        trash0 = OUT_ROWS + sid * (BLK // 16)

        def drain_scatter(rslot):
            pltpu.make_async_copy(t_hbm.at[pl.ds(0, BLK), :],
                                  rows.at[rslot], sems[rslot]).wait()
            pltpu.sync_copy(rows.at[rslot], acc_sh.at[idxw.at[rslot]],
                            add=True)
            pltpu.sync_copy(ones_v, cnt_sh.at[idxw.at[rslot]], add=True)

        def fire_block(eslot, b, rslot, drain_prev):
            for j in range(BLK // 16):
                d = ebuf[eslot, b, pl.ds(BLK + j * 16, 16)]
                local = d - base
                ok = (local >= 0) & (local < OUT_ROWS)
                idxw[rslot, pl.ds(j * 16, 16)] = jnp.where(
                    ok, local, trash0 + j)
            pltpu.async_copy(t_hbm.at[ebuf.at[eslot, b, pl.ds(0, BLK)]],
                             rows.at[rslot], sems[rslot])
            if drain_prev:
                drain_scatter(1 - rslot)

        def run_iter(first_line, skip_first_drain):
            pltpu.sync_copy(eib_hbm.at[pl.ds(first_line, GRP), :],
                            ebuf.at[0])
            for j in range(GRP):
                fire_block(0, j, j % 2,
                           drain_prev=not (skip_first_drain and j == 0))
            pltpu.sync_copy(eib_hbm.at[pl.ds(first_line + GRP, GRP), :],
                            ebuf.at[1])
            for j in range(GRP, ITER_BLKS):
                fire_block(1, j - GRP, j % 2, drain_prev=True)

        run_iter(line0, skip_first_drain=True)
        def body(ii, _):
            run_iter(line0 + ITER_BLKS * ii, False)
            return 0
        if n_iter > 1:
            lax.fori_loop(1, n_iter, body, 0)
        drain_scatter((ITER_BLKS - 1) % 2)
        plsc.subcore_barrier()

        o0 = sid * PER_TILE_OUT
        pltpu.sync_copy(acc_sh.at[pl.ds(o0, PER_TILE_OUT), :],
                        out_hbm.at[cid, pl.ds(o0, PER_TILE_OUT), :])
        pltpu.sync_copy(cnt_sh.at[pl.ds(o0, PER_TILE_OUT)],
                        zflat.at[pl.ds(0, PER_TILE_OUT)])
        pltpu.sync_copy(zflat.at[pl.ds(0, PER_TILE_OUT)],
                        cnt_hbm.at[pl.ds(cid * OUT_ROWS + o0, PER_TILE_OUT)])

    return agg


def _pad_edges(ei, split_edges):
    e = ei.shape[1]
    mult = ITER_BLKS * BLK * (32 if split_edges else 16)
    e_pad = ((e + mult - 1) // mult) * mult
    src = jnp.concatenate(
        [ei[0], jnp.zeros((e_pad - e,), jnp.int32)])
    dst = jnp.concatenate(
        [ei[1], jnp.full((e_pad - e,), PAD_DST, jnp.int32)])
    eib = jnp.concatenate(
        [src.reshape(-1, BLK), dst.reshape(-1, BLK)], axis=1)
    return eib, e_pad


def _assemble(parts, idx_bound, n_dst, split_edges):
    if split_edges:
        full = parts[0] + parts[1]
        full = full[:min(idx_bound, n_dst)]
    else:
        full = jnp.concatenate([parts[0], parts[1]], axis=0)[:n_dst]
    if full.shape[0] < n_dst:
        pad = [(0, n_dst - full.shape[0])] + [(0, 0)] * (full.ndim - 1)
        full = jnp.pad(full, pad)
    return full


def _enc_body(x_ref, w1t_ref, b1_ref, w2t_ref, b2_ref, o_ref):
    h = jnp.maximum(
        jnp.dot(x_ref[...], w1t_ref[...], preferred_element_type=jnp.float32)
        + b1_ref[...], 0.0)
    o_ref[...] = (
        jnp.dot(h, w2t_ref[...], preferred_element_type=jnp.float32)
        + b2_ref[...])


def _encoder_mlp(p, x, blk=512):
    n, in_dim = x.shape
    hid = p["W1"].shape[0]
    out_dim = p["W2"].shape[0]
    n_pad = ((n + blk - 1) // blk) * blk
    if n_pad != n:
        x = jnp.pad(x, ((0, n_pad - n), (0, 0)))
    out = pl.pallas_call(
        _enc_body,
        grid=(n_pad // blk,),
        in_specs=[
            pl.BlockSpec((blk, in_dim), lambda i: (i, 0)),
            pl.BlockSpec((in_dim, hid), lambda i: (0, 0)),
            pl.BlockSpec((1, hid), lambda i: (0, 0)),
            pl.BlockSpec((hid, out_dim), lambda i: (0, 0)),
            pl.BlockSpec((1, out_dim), lambda i: (0, 0)),
        ],
        out_specs=pl.BlockSpec((blk, out_dim), lambda i: (i, 0)),
        out_shape=jax.ShapeDtypeStruct((n_pad, out_dim), jnp.float32),
    )(x, p["W1"].T, p["b1"][None, :], p["W2"].T, p["b2"][None, :])
    return out[:n]


def _layer_norm(n, x):
    m = x.mean(-1, keepdims=True)
    v = jnp.mean((x - m) ** 2, axis=-1, keepdims=True)
    return (x - m) / jnp.sqrt(v + 1e-5) * n["g"] + n["b"]


def kernel(point_x, edge_x, face_x, ei_pp, ei_fp, ei_ep, ei_pf, ei_ef, ei_ff,
           ei_pe, ei_fe, batch_point, batch_edge, batch_face, params):
    ei = {"pp": ei_pp, "fp": ei_fp, "ep": ei_ep, "pf": ei_pf, "ef": ei_ef,
          "ff": ei_ff, "pe": ei_pe, "fe": ei_fe}
    h = {
        "p": _encoder_mlp(params["enc_point"], point_x),
        "e": _encoder_mlp(params["enc_edge"], edge_x),
        "f": _encoder_mlp(params["enc_face"], face_x),
    }
    n_nodes = {"p": point_x.shape[0], "e": edge_x.shape[0],
               "f": face_x.shape[0]}
    tname = {"p": "point", "e": "edge", "f": "face"}

    prep = {}
    for r in REL_LIST:
        s_t, d_t = REL_SRC_DST[r]
        idx_bound = min(n_nodes[s_t], n_nodes[d_t])
        split_edges = idx_bound <= OUT_ROWS
        eib, e_pad = _pad_edges(ei[r], split_edges)
        prep[r] = (eib, e_pad, split_edges, idx_bound)

    inv_cnt = {}
    for i in range(2):
        cp = params["convs"][i]
        out = {}
        for t in ["p", "f", "e"]:
            rels = [r for r in REL_LIST if REL_SRC_DST[r][1] == t]
            wr_sum = sum(cp[r]["Wr"] for r in rels)
            bl_sum = sum(cp[r]["bl"] for r in rels)
            acc = h[t] @ wr_sum.T + bl_sum
            for r in rels:
                s_t = REL_SRC_DST[r][0]
                eib, e_pad, split_edges, idx_bound = prep[r]
                T = h[s_t] @ cp[r]["Wl"].T
                agg2, cnt_flat = _make_agg(e_pad, split_edges)(eib, T)
                if i == 0:
                    cnt = _assemble(cnt_flat.reshape(2, OUT_ROWS), idx_bound,
                                    n_nodes[t], split_edges)
                    inv_cnt[r] = 1.0 / jnp.maximum(cnt, 1.0)
                agg = _assemble(agg2, idx_bound, n_nodes[t], split_edges)
                acc = acc + agg * inv_cnt[r][:, None]
            out[t] = acc
        nrm = params["norms"][i]
        for t in ["p", "f", "e"]:
            h[t] = _layer_norm(nrm[tname[t]], h[t] + jnp.maximum(out[t], 0.0))

    def _pool(x, batch):
        s = jax.ops.segment_sum(x, batch, num_segments=16)
        c = jax.ops.segment_sum(
            jnp.ones((x.shape[0],), jnp.float32), batch, num_segments=16)
        return s / jnp.maximum(c, 1.0)[:, None]

    g_p = _pool(h["p"], batch_point)
    g_f = _pool(h["f"], batch_face)
    g_e = _pool(h["e"], batch_edge)
    return jnp.concatenate([g_p, g_e, g_f], axis=-1)


# speedup vs baseline: 1.0234x; 1.0234x over previous
import functools

import jax
import jax.numpy as jnp
from jax import lax
from jax.experimental import pallas as pl
from jax.experimental.pallas import tpu as pltpu
from jax.experimental.pallas import tpu_sc as plsc

REL_LIST = ["pp", "fp", "ep", "pf", "ef", "ff", "pe", "fe"]
REL_SRC_DST = {
    "pp": ("p", "p"), "fp": ("f", "p"), "ep": ("e", "p"),
    "pf": ("p", "f"), "ef": ("e", "f"), "ff": ("f", "f"),
    "pe": ("p", "e"), "fe": ("f", "e"),
}

HID = 64
BLK = 128
G = 4
GB = G * BLK
OUT_ROWS = 25088
TRASH = 128
ACC_ROWS = OUT_ROWS + TRASH
PER_TILE_ACC = ACC_ROWS // 16
PER_TILE_OUT = OUT_ROWS // 16
PAD_DST = 1 << 29

_MESH = dict(core_axis_name="c", subcore_axis_name="s", num_cores=2,
             num_subcores=16)
_SC_PARAMS = pltpu.CompilerParams(use_tc_tiling_on_sc=False)


GRP = 4
ITER_BLKS = 2 * GRP


def _make_agg(e_pad, split_edges, has_counts=True):
    n_tiles = 32 if split_edges else 16
    chunk = e_pad // n_tiles
    n_blk = chunk // BLK
    assert n_blk % ITER_BLKS == 0
    n_iter = n_blk // ITER_BLKS
    mesh = plsc.VectorSubcoreMesh(**_MESH)

    out_type = [jax.ShapeDtypeStruct((2, OUT_ROWS, HID), jnp.float32)]
    if has_counts:
        out_type.append(jax.ShapeDtypeStruct((2 * OUT_ROWS,), jnp.float32))

    @functools.partial(
        pl.kernel,
        out_type=tuple(out_type),
        mesh=mesh,
        compiler_params=_SC_PARAMS,
        scratch_types=[
            pltpu.VMEM((2, GRP, 2 * BLK), jnp.int32),
            pltpu.VMEM((2, BLK), jnp.int32),
            pltpu.VMEM((2, BLK, HID), jnp.float32),
            pltpu.VMEM((BLK,), jnp.float32),
            pltpu.VMEM((PER_TILE_ACC,), jnp.float32),
            pltpu.VMEM_SHARED((ACC_ROWS, HID), jnp.float32),
            pltpu.VMEM_SHARED((ACC_ROWS,), jnp.float32),
            pltpu.SemaphoreType.DMA,
            pltpu.SemaphoreType.DMA,
        ],
    )
    def agg(eib_hbm, t_hbm, *refs):
        if has_counts:
            (out_hbm, cnt_hbm, ebuf, idxw, rows, ones_v,
             zflat, acc_sh, cnt_sh, sem0, sem1) = refs
        else:
            (out_hbm, ebuf, idxw, rows, ones_v,
             zflat, acc_sh, cnt_sh, sem0, sem1) = refs
        cid = lax.axis_index("c")
        sid = lax.axis_index("s")
        sems = (sem0, sem1)

        def zrow(r, _):
            for k in range(HID // 16):
                rows[0, r, pl.ds(k * 16, 16)] = jnp.zeros((16,), jnp.float32)
            return 0
        lax.fori_loop(0, BLK, zrow, 0)
        for j in range(BLK // 16):
            ones_v[pl.ds(j * 16, 16)] = jnp.ones((16,), jnp.float32)
        def zf(i, _):
            zflat[pl.ds(i * 16, 16)] = jnp.zeros((16,), jnp.float32)
            return 0
        lax.fori_loop(0, PER_TILE_ACC // 16, zf, 0)
        if PER_TILE_ACC % 16:
            zflat[pl.ds(PER_TILE_ACC - 16, 16)] = jnp.zeros((16,),
                                                            jnp.float32)
        r0 = sid * PER_TILE_ACC
        def zacc(k, _):
            pltpu.sync_copy(rows.at[0],
                            acc_sh.at[pl.ds(r0 + k * BLK, BLK), :])
            return 0
        lax.fori_loop(0, PER_TILE_ACC // BLK, zacc, 0)
        rem = PER_TILE_ACC % BLK
        if rem:
            pltpu.sync_copy(
                rows.at[0, pl.ds(0, rem), :],
                acc_sh.at[pl.ds(r0 + PER_TILE_ACC - rem, rem), :])
        pltpu.sync_copy(zflat, cnt_sh.at[pl.ds(r0, PER_TILE_ACC)])
        plsc.subcore_barrier()

        if split_edges:
            base = jnp.int32(0)
            tid = cid * 16 + sid
        else:
            base = cid * OUT_ROWS
            tid = sid
        line0 = tid * n_blk
        trash0 = OUT_ROWS + sid * (BLK // 16)

        def drain_scatter(rslot):
            pltpu.make_async_copy(t_hbm.at[pl.ds(0, BLK), :],
                                  rows.at[rslot], sems[rslot]).wait()
            pltpu.sync_copy(rows.at[rslot], acc_sh.at[idxw.at[rslot]],
                            add=True)
            if has_counts:
                pltpu.sync_copy(ones_v, cnt_sh.at[idxw.at[rslot]], add=True)

        def fire_block(eslot, b, rslot, drain_prev):
            for j in range(BLK // 16):
                d = ebuf[eslot, b, pl.ds(BLK + j * 16, 16)]
                local = d - base
                ok = (local >= 0) & (local < OUT_ROWS)
                idxw[rslot, pl.ds(j * 16, 16)] = jnp.where(
                    ok, local, trash0 + j)
            pltpu.async_copy(t_hbm.at[ebuf.at[eslot, b, pl.ds(0, BLK)]],
                             rows.at[rslot], sems[rslot])
            if drain_prev:
                drain_scatter(1 - rslot)

        def run_iter(first_line, skip_first_drain):
            pltpu.sync_copy(eib_hbm.at[pl.ds(first_line, GRP), :],
                            ebuf.at[0])
            for j in range(GRP):
                fire_block(0, j, j % 2,
                           drain_prev=not (skip_first_drain and j == 0))
            pltpu.sync_copy(eib_hbm.at[pl.ds(first_line + GRP, GRP), :],
                            ebuf.at[1])
            for j in range(GRP, ITER_BLKS):
                fire_block(1, j - GRP, j % 2, drain_prev=True)

        run_iter(line0, skip_first_drain=True)
        def body(ii, _):
            run_iter(line0 + ITER_BLKS * ii, False)
            return 0
        if n_iter > 1:
            lax.fori_loop(1, n_iter, body, 0)
        drain_scatter((ITER_BLKS - 1) % 2)
        plsc.subcore_barrier()

        o0 = sid * PER_TILE_OUT
        pltpu.sync_copy(acc_sh.at[pl.ds(o0, PER_TILE_OUT), :],
                        out_hbm.at[cid, pl.ds(o0, PER_TILE_OUT), :])
        if has_counts:
            pltpu.sync_copy(cnt_sh.at[pl.ds(o0, PER_TILE_OUT)],
                            zflat.at[pl.ds(0, PER_TILE_OUT)])
            pltpu.sync_copy(
                zflat.at[pl.ds(0, PER_TILE_OUT)],
                cnt_hbm.at[pl.ds(cid * OUT_ROWS + o0, PER_TILE_OUT)])

    return agg


def _pad_edges(ei, split_edges):
    e = ei.shape[1]
    mult = ITER_BLKS * BLK * (32 if split_edges else 16)
    e_pad = ((e + mult - 1) // mult) * mult
    src = jnp.concatenate(
        [ei[0], jnp.zeros((e_pad - e,), jnp.int32)])
    dst = jnp.concatenate(
        [ei[1], jnp.full((e_pad - e,), PAD_DST, jnp.int32)])
    eib = jnp.concatenate(
        [src.reshape(-1, BLK), dst.reshape(-1, BLK)], axis=1)
    return eib, e_pad


def _assemble(parts, idx_bound, n_dst, split_edges):
    if split_edges:
        full = parts[0] + parts[1]
        full = full[:min(idx_bound, n_dst)]
    else:
        full = jnp.concatenate([parts[0], parts[1]], axis=0)[:n_dst]
    if full.shape[0] < n_dst:
        pad = [(0, n_dst - full.shape[0])] + [(0, 0)] * (full.ndim - 1)
        full = jnp.pad(full, pad)
    return full


def _enc_body(x_ref, w1t_ref, b1_ref, w2t_ref, b2_ref, o_ref):
    h = jnp.maximum(
        jnp.dot(x_ref[...], w1t_ref[...], preferred_element_type=jnp.float32)
        + b1_ref[...], 0.0)
    o_ref[...] = (
        jnp.dot(h, w2t_ref[...], preferred_element_type=jnp.float32)
        + b2_ref[...])


def _encoder_mlp(p, x, blk=512):
    n, in_dim = x.shape
    hid = p["W1"].shape[0]
    out_dim = p["W2"].shape[0]
    n_pad = ((n + blk - 1) // blk) * blk
    if n_pad != n:
        x = jnp.pad(x, ((0, n_pad - n), (0, 0)))
    out = pl.pallas_call(
        _enc_body,
        grid=(n_pad // blk,),
        in_specs=[
            pl.BlockSpec((blk, in_dim), lambda i: (i, 0)),
            pl.BlockSpec((in_dim, hid), lambda i: (0, 0)),
            pl.BlockSpec((1, hid), lambda i: (0, 0)),
            pl.BlockSpec((hid, out_dim), lambda i: (0, 0)),
            pl.BlockSpec((1, out_dim), lambda i: (0, 0)),
        ],
        out_specs=pl.BlockSpec((blk, out_dim), lambda i: (i, 0)),
        out_shape=jax.ShapeDtypeStruct((n_pad, out_dim), jnp.float32),
    )(x, p["W1"].T, p["b1"][None, :], p["W2"].T, p["b2"][None, :])
    return out[:n]


def _layer_norm(n, x):
    m = x.mean(-1, keepdims=True)
    v = jnp.mean((x - m) ** 2, axis=-1, keepdims=True)
    return (x - m) / jnp.sqrt(v + 1e-5) * n["g"] + n["b"]


def kernel(point_x, edge_x, face_x, ei_pp, ei_fp, ei_ep, ei_pf, ei_ef, ei_ff,
           ei_pe, ei_fe, batch_point, batch_edge, batch_face, params):
    ei = {"pp": ei_pp, "fp": ei_fp, "ep": ei_ep, "pf": ei_pf, "ef": ei_ef,
          "ff": ei_ff, "pe": ei_pe, "fe": ei_fe}
    h = {
        "p": _encoder_mlp(params["enc_point"], point_x),
        "e": _encoder_mlp(params["enc_edge"], edge_x),
        "f": _encoder_mlp(params["enc_face"], face_x),
    }
    n_nodes = {"p": point_x.shape[0], "e": edge_x.shape[0],
               "f": face_x.shape[0]}
    tname = {"p": "point", "e": "edge", "f": "face"}

    prep = {}
    for r in REL_LIST:
        s_t, d_t = REL_SRC_DST[r]
        idx_bound = min(n_nodes[s_t], n_nodes[d_t])
        split_edges = idx_bound <= OUT_ROWS
        eib, e_pad = _pad_edges(ei[r], split_edges)
        prep[r] = (eib, e_pad, split_edges, idx_bound)

    inv_cnt = {}
    for i in range(2):
        cp = params["convs"][i]
        out = {}
        for t in ["p", "f", "e"]:
            rels = [r for r in REL_LIST if REL_SRC_DST[r][1] == t]
            wr_sum = sum(cp[r]["Wr"] for r in rels)
            bl_sum = sum(cp[r]["bl"] for r in rels)
            acc = h[t] @ wr_sum.T + bl_sum
            for r in rels:
                s_t = REL_SRC_DST[r][0]
                eib, e_pad, split_edges, idx_bound = prep[r]
                T = h[s_t] @ cp[r]["Wl"].T
                if i == 0:
                    agg2, cnt_flat = _make_agg(e_pad, split_edges)(eib, T)
                    cnt = _assemble(cnt_flat.reshape(2, OUT_ROWS), idx_bound,
                                    n_nodes[t], split_edges)
                    inv_cnt[r] = 1.0 / jnp.maximum(cnt, 1.0)
                else:
                    (agg2,) = _make_agg(e_pad, split_edges,
                                        has_counts=False)(eib, T)
                agg = _assemble(agg2, idx_bound, n_nodes[t], split_edges)
                acc = acc + agg * inv_cnt[r][:, None]
            out[t] = acc
        nrm = params["norms"][i]
        for t in ["p", "f", "e"]:
            h[t] = _layer_norm(nrm[tname[t]], h[t] + jnp.maximum(out[t], 0.0))

    def _pool(x, batch):
        s = jax.ops.segment_sum(x, batch, num_segments=16)
        c = jax.ops.segment_sum(
            jnp.ones((x.shape[0],), jnp.float32), batch, num_segments=16)
        return s / jnp.maximum(c, 1.0)[:, None]

    g_p = _pool(h["p"], batch_point)
    g_f = _pool(h["f"], batch_face)
    g_e = _pool(h["e"], batch_edge)
    return jnp.concatenate([g_p, g_e, g_f], axis=-1)


# speedup vs baseline: 1.0314x; 1.0078x over previous
import functools

import jax
import jax.numpy as jnp
from jax import lax
from jax.experimental import pallas as pl
from jax.experimental.pallas import tpu as pltpu
from jax.experimental.pallas import tpu_sc as plsc

REL_LIST = ["pp", "fp", "ep", "pf", "ef", "ff", "pe", "fe"]
REL_SRC_DST = {
    "pp": ("p", "p"), "fp": ("f", "p"), "ep": ("e", "p"),
    "pf": ("p", "f"), "ef": ("e", "f"), "ff": ("f", "f"),
    "pe": ("p", "e"), "fe": ("f", "e"),
}

HID = 64
BLK = 128
G = 4
GB = G * BLK
OUT_ROWS = 25088
TRASH = 128
ACC_ROWS = OUT_ROWS + TRASH
PER_TILE_ACC = ACC_ROWS // 16
PER_TILE_OUT = OUT_ROWS // 16
PAD_DST = 1 << 29

_MESH = dict(core_axis_name="c", subcore_axis_name="s", num_cores=2,
             num_subcores=16)
_SC_PARAMS = pltpu.CompilerParams(use_tc_tiling_on_sc=False)


GRP = 4
ITER_BLKS = 2 * GRP


def _make_agg(e_pad, split_edges, has_counts=True):
    n_tiles = 32 if split_edges else 16
    chunk = e_pad // n_tiles
    n_blk = chunk // BLK
    assert n_blk % ITER_BLKS == 0
    n_iter = n_blk // ITER_BLKS
    mesh = plsc.VectorSubcoreMesh(**_MESH)

    out_type = [jax.ShapeDtypeStruct((2, OUT_ROWS, HID), jnp.float32)]
    if has_counts:
        out_type.append(jax.ShapeDtypeStruct((2 * OUT_ROWS,), jnp.float32))

    @functools.partial(
        pl.kernel,
        out_type=tuple(out_type),
        mesh=mesh,
        compiler_params=_SC_PARAMS,
        scratch_types=[
            pltpu.VMEM((2, GRP, 2 * BLK), jnp.int32),
            pltpu.VMEM((2, BLK), jnp.int32),
            pltpu.VMEM((2, BLK), jnp.int32),
            pltpu.VMEM((2, BLK, HID), jnp.float32),
            pltpu.VMEM((BLK,), jnp.float32),
            pltpu.VMEM((PER_TILE_ACC,), jnp.float32),
            pltpu.VMEM_SHARED((ACC_ROWS, HID), jnp.float32),
            pltpu.VMEM_SHARED((ACC_ROWS,), jnp.float32),
            pltpu.SemaphoreType.DMA,
            pltpu.SemaphoreType.DMA,
            pltpu.SemaphoreType.DMA,
            pltpu.SemaphoreType.DMA,
            pltpu.SemaphoreType.DMA,
            pltpu.SemaphoreType.DMA,
        ],
    )
    def agg(eib_hbm, t_hbm, *refs):
        if has_counts:
            (out_hbm, cnt_hbm, ebuf, srcw, idxw, rows, ones_v, zflat,
             acc_sh, cnt_sh, g0, g1, s0, s1, c0, c1) = refs
        else:
            (out_hbm, ebuf, srcw, idxw, rows, ones_v, zflat,
             acc_sh, cnt_sh, g0, g1, s0, s1, c0, c1) = refs
        cid = lax.axis_index("c")
        sid = lax.axis_index("s")
        gsems = (g0, g1)
        ssems = (s0, s1)
        csems = (c0, c1)

        def zrow(r, _):
            for k in range(HID // 16):
                rows[0, r, pl.ds(k * 16, 16)] = jnp.zeros((16,), jnp.float32)
            return 0
        lax.fori_loop(0, BLK, zrow, 0)
        for j in range(BLK // 16):
            ones_v[pl.ds(j * 16, 16)] = jnp.ones((16,), jnp.float32)
        def zf(i, _):
            zflat[pl.ds(i * 16, 16)] = jnp.zeros((16,), jnp.float32)
            return 0
        lax.fori_loop(0, PER_TILE_ACC // 16, zf, 0)
        if PER_TILE_ACC % 16:
            zflat[pl.ds(PER_TILE_ACC - 16, 16)] = jnp.zeros((16,),
                                                            jnp.float32)
        r0 = sid * PER_TILE_ACC
        def zacc(k, _):
            pltpu.sync_copy(rows.at[0],
                            acc_sh.at[pl.ds(r0 + k * BLK, BLK), :])
            return 0
        lax.fori_loop(0, PER_TILE_ACC // BLK, zacc, 0)
        rem = PER_TILE_ACC % BLK
        if rem:
            pltpu.sync_copy(
                rows.at[0, pl.ds(0, rem), :],
                acc_sh.at[pl.ds(r0 + PER_TILE_ACC - rem, rem), :])
        pltpu.sync_copy(zflat, cnt_sh.at[pl.ds(r0, PER_TILE_ACC)])
        plsc.subcore_barrier()

        if split_edges:
            base = jnp.int32(0)
            tid = cid * 16 + sid
        else:
            base = cid * OUT_ROWS
            tid = sid
        line0 = tid * n_blk
        trash0 = OUT_ROWS + sid * (BLK // 16)

        def drain_scatters(rslot):
            pltpu.make_async_copy(rows.at[rslot],
                                  acc_sh.at[idxw.at[rslot]],
                                  ssems[rslot]).wait()
            if has_counts:
                pltpu.make_async_copy(ones_v, cnt_sh.at[idxw.at[rslot]],
                                      csems[rslot]).wait()

        def finish_prev(rslot):
            pltpu.make_async_copy(t_hbm.at[pl.ds(0, BLK), :],
                                  rows.at[rslot], gsems[rslot]).wait()
            pltpu.async_copy(rows.at[rslot], acc_sh.at[idxw.at[rslot]],
                             ssems[rslot], add=True)
            if has_counts:
                pltpu.async_copy(ones_v, cnt_sh.at[idxw.at[rslot]],
                                 csems[rslot], add=True)

        def fire_block(eslot, b, rslot, drain_prev, drain_scat):
            if drain_scat:
                drain_scatters(rslot)
            for j in range(BLK // 16):
                s = ebuf[eslot, b, pl.ds(j * 16, 16)]
                srcw[rslot, pl.ds(j * 16, 16)] = s
                d = ebuf[eslot, b, pl.ds(BLK + j * 16, 16)]
                local = d - base
                ok = (local >= 0) & (local < OUT_ROWS)
                idxw[rslot, pl.ds(j * 16, 16)] = jnp.where(
                    ok, local, trash0 + j)
            pltpu.async_copy(t_hbm.at[srcw.at[rslot]],
                             rows.at[rslot], gsems[rslot])
            if drain_prev:
                finish_prev(1 - rslot)

        def run_iter(first_line, is_first):
            pltpu.sync_copy(eib_hbm.at[pl.ds(first_line, GRP), :],
                            ebuf.at[0])
            for j in range(GRP):
                fire_block(0, j, j % 2,
                           drain_prev=not (is_first and j == 0),
                           drain_scat=not (is_first and j < 2))
            pltpu.sync_copy(eib_hbm.at[pl.ds(first_line + GRP, GRP), :],
                            ebuf.at[1])
            for j in range(GRP, ITER_BLKS):
                fire_block(1, j - GRP, j % 2, drain_prev=True,
                           drain_scat=True)

        run_iter(line0, is_first=True)
        def body(ii, _):
            run_iter(line0 + ITER_BLKS * ii, False)
            return 0
        if n_iter > 1:
            lax.fori_loop(1, n_iter, body, 0)
        finish_prev((ITER_BLKS - 1) % 2)
        drain_scatters(0)
        drain_scatters(1)
        plsc.subcore_barrier()

        o0 = sid * PER_TILE_OUT
        pltpu.sync_copy(acc_sh.at[pl.ds(o0, PER_TILE_OUT), :],
                        out_hbm.at[cid, pl.ds(o0, PER_TILE_OUT), :])
        if has_counts:
            pltpu.sync_copy(cnt_sh.at[pl.ds(o0, PER_TILE_OUT)],
                            zflat.at[pl.ds(0, PER_TILE_OUT)])
            pltpu.sync_copy(
                zflat.at[pl.ds(0, PER_TILE_OUT)],
                cnt_hbm.at[pl.ds(cid * OUT_ROWS + o0, PER_TILE_OUT)])

    return agg


def _pad_edges(ei, split_edges):
    e = ei.shape[1]
    mult = ITER_BLKS * BLK * (32 if split_edges else 16)
    e_pad = ((e + mult - 1) // mult) * mult
    src = jnp.concatenate(
        [ei[0], jnp.zeros((e_pad - e,), jnp.int32)])
    dst = jnp.concatenate(
        [ei[1], jnp.full((e_pad - e,), PAD_DST, jnp.int32)])
    eib = jnp.concatenate(
        [src.reshape(-1, BLK), dst.reshape(-1, BLK)], axis=1)
    return eib, e_pad


def _assemble(parts, idx_bound, n_dst, split_edges):
    if split_edges:
        full = parts[0] + parts[1]
        full = full[:min(idx_bound, n_dst)]
    else:
        full = jnp.concatenate([parts[0], parts[1]], axis=0)[:n_dst]
    if full.shape[0] < n_dst:
        pad = [(0, n_dst - full.shape[0])] + [(0, 0)] * (full.ndim - 1)
        full = jnp.pad(full, pad)
    return full


def _enc_body(x_ref, w1t_ref, b1_ref, w2t_ref, b2_ref, o_ref):
    h = jnp.maximum(
        jnp.dot(x_ref[...], w1t_ref[...], preferred_element_type=jnp.float32)
        + b1_ref[...], 0.0)
    o_ref[...] = (
        jnp.dot(h, w2t_ref[...], preferred_element_type=jnp.float32)
        + b2_ref[...])


def _encoder_mlp(p, x, blk=512):
    n, in_dim = x.shape
    hid = p["W1"].shape[0]
    out_dim = p["W2"].shape[0]
    n_pad = ((n + blk - 1) // blk) * blk
    if n_pad != n:
        x = jnp.pad(x, ((0, n_pad - n), (0, 0)))
    out = pl.pallas_call(
        _enc_body,
        grid=(n_pad // blk,),
        in_specs=[
            pl.BlockSpec((blk, in_dim), lambda i: (i, 0)),
            pl.BlockSpec((in_dim, hid), lambda i: (0, 0)),
            pl.BlockSpec((1, hid), lambda i: (0, 0)),
            pl.BlockSpec((hid, out_dim), lambda i: (0, 0)),
            pl.BlockSpec((1, out_dim), lambda i: (0, 0)),
        ],
        out_specs=pl.BlockSpec((blk, out_dim), lambda i: (i, 0)),
        out_shape=jax.ShapeDtypeStruct((n_pad, out_dim), jnp.float32),
    )(x, p["W1"].T, p["b1"][None, :], p["W2"].T, p["b2"][None, :])
    return out[:n]


def _layer_norm(n, x):
    m = x.mean(-1, keepdims=True)
    v = jnp.mean((x - m) ** 2, axis=-1, keepdims=True)
    return (x - m) / jnp.sqrt(v + 1e-5) * n["g"] + n["b"]


def kernel(point_x, edge_x, face_x, ei_pp, ei_fp, ei_ep, ei_pf, ei_ef, ei_ff,
           ei_pe, ei_fe, batch_point, batch_edge, batch_face, params):
    ei = {"pp": ei_pp, "fp": ei_fp, "ep": ei_ep, "pf": ei_pf, "ef": ei_ef,
          "ff": ei_ff, "pe": ei_pe, "fe": ei_fe}
    h = {
        "p": _encoder_mlp(params["enc_point"], point_x),
        "e": _encoder_mlp(params["enc_edge"], edge_x),
        "f": _encoder_mlp(params["enc_face"], face_x),
    }
    n_nodes = {"p": point_x.shape[0], "e": edge_x.shape[0],
               "f": face_x.shape[0]}
    tname = {"p": "point", "e": "edge", "f": "face"}

    prep = {}
    for r in REL_LIST:
        s_t, d_t = REL_SRC_DST[r]
        idx_bound = min(n_nodes[s_t], n_nodes[d_t])
        split_edges = idx_bound <= OUT_ROWS
        eib, e_pad = _pad_edges(ei[r], split_edges)
        prep[r] = (eib, e_pad, split_edges, idx_bound)

    inv_cnt = {}
    for i in range(2):
        cp = params["convs"][i]
        out = {}
        for t in ["p", "f", "e"]:
            rels = [r for r in REL_LIST if REL_SRC_DST[r][1] == t]
            wr_sum = sum(cp[r]["Wr"] for r in rels)
            bl_sum = sum(cp[r]["bl"] for r in rels)
            acc = h[t] @ wr_sum.T + bl_sum
            for r in rels:
                s_t = REL_SRC_DST[r][0]
                eib, e_pad, split_edges, idx_bound = prep[r]
                T = h[s_t] @ cp[r]["Wl"].T
                if i == 0:
                    agg2, cnt_flat = _make_agg(e_pad, split_edges)(eib, T)
                    cnt = _assemble(cnt_flat.reshape(2, OUT_ROWS), idx_bound,
                                    n_nodes[t], split_edges)
                    inv_cnt[r] = 1.0 / jnp.maximum(cnt, 1.0)
                else:
                    (agg2,) = _make_agg(e_pad, split_edges,
                                        has_counts=False)(eib, T)
                agg = _assemble(agg2, idx_bound, n_nodes[t], split_edges)
                acc = acc + agg * inv_cnt[r][:, None]
            out[t] = acc
        nrm = params["norms"][i]
        for t in ["p", "f", "e"]:
            h[t] = _layer_norm(nrm[tname[t]], h[t] + jnp.maximum(out[t], 0.0))

    def _pool(x, batch):
        s = jax.ops.segment_sum(x, batch, num_segments=16)
        c = jax.ops.segment_sum(
            jnp.ones((x.shape[0],), jnp.float32), batch, num_segments=16)
        return s / jnp.maximum(c, 1.0)[:, None]

    g_p = _pool(h["p"], batch_point)
    g_f = _pool(h["f"], batch_face)
    g_e = _pool(h["e"], batch_edge)
    return jnp.concatenate([g_p, g_e, g_f], axis=-1)
